# Initial kernel scaffold; baseline (speedup 1.0000x reference)
import functools

import jax
import jax.numpy as jnp
from jax import lax
from jax.experimental import pallas as pl
from jax.experimental.pallas import tpu as pltpu
from jax.experimental.pallas import tpu_sc as plsc

N_NODES = 10000
N_EDGES = 320000
D = 128
NC, NS = 2, 16
NW = NC * NS
C = 128
NCHUNK = 79
EPW = NCHUNK * C
E_PAD = NW * EPW
N_PAD = 10112
RPT = N_PAD // NS
CW = 16

_ZERO16 = jnp.zeros((16,), jnp.float32)
_ONE16 = jnp.ones((16,), jnp.float32)


def _make_agg(do_count: bool):
  mesh = plsc.VectorSubcoreMesh(core_axis_name="c", subcore_axis_name="s")
  out_type = [jax.ShapeDtypeStruct((NC, N_PAD, D), jnp.float32)]
  scratch = [
      pltpu.VMEM((C,), jnp.int32),
      pltpu.VMEM((C,), jnp.int32),
      pltpu.VMEM((C, D), jnp.float32),
      pltpu.VMEM((NCHUNK, D), jnp.float32),
      pltpu.VMEM_SHARED((N_PAD, D), jnp.float32),
      pltpu.SemaphoreType.DMA,
  ]
  if do_count:
    out_type.append(jax.ShapeDtypeStruct((NC, N_PAD, CW), jnp.float32))
    scratch += [
        pltpu.VMEM((C, CW), jnp.float32),
        pltpu.VMEM((NCHUNK, CW), jnp.float32),
        pltpu.VMEM_SHARED((N_PAD, CW), jnp.float32),
    ]

  def body(x_hbm, src_hbm, dst_hbm, *rest):
    if do_count:
      (acc_out, cnt_out, src_v, dst_v, rows_v, zrow_v, acc_sh, sem,
       ones_v, zc_v, cnt_sh) = rest
    else:
      acc_out, src_v, dst_v, rows_v, zrow_v, acc_sh, sem = rest

    c = lax.axis_index("c")
    s = lax.axis_index("s")
    w = c * NS + s
    base = s * RPT

    def zrow_body(i, carry):
      for j in range(D // 16):
        zrow_v[i, pl.ds(j * 16, 16)] = _ZERO16
      if do_count:
        zc_v[i, pl.ds(0, CW)] = _ZERO16
      return carry

    lax.fori_loop(0, NCHUNK, zrow_body, 0)
    if do_count:
      def ones_body(i, carry):
        ones_v[i, pl.ds(0, CW)] = _ONE16
        return carry
      lax.fori_loop(0, C, ones_body, 0)
    for k in range(RPT // NCHUNK):
      pltpu.sync_copy(zrow_v, acc_sh.at[pl.ds(base + k * NCHUNK, NCHUNK)])
      if do_count:
        pltpu.sync_copy(zc_v, cnt_sh.at[pl.ds(base + k * NCHUNK, NCHUNK)])
    plsc.subcore_barrier()

    def edge_body(j, carry):
      pltpu.sync_copy(src_hbm.at[w, j], src_v)
      pltpu.sync_copy(dst_hbm.at[w, j], dst_v)
      pltpu.async_copy(x_hbm.at[src_v], rows_v, sem).wait()
      pltpu.sync_copy(rows_v, acc_sh.at[dst_v], add=True)
      if do_count:
        pltpu.sync_copy(ones_v, cnt_sh.at[dst_v], add=True)
      return carry

    lax.fori_loop(0, NCHUNK, edge_body, 0)
    plsc.subcore_barrier()

    pltpu.sync_copy(acc_sh.at[pl.ds(base, RPT)], acc_out.at[c, pl.ds(base, RPT)])
    if do_count:
      pltpu.sync_copy(cnt_sh.at[pl.ds(base, RPT)], cnt_out.at[c, pl.ds(base, RPT)])

  return pl.kernel(body, mesh=mesh, out_type=tuple(out_type),
                   scratch_types=tuple(scratch))


_agg_count = _make_agg(True)
_agg_plain = _make_agg(False)


def _make_dense(relu: bool):
  R = 632

  def body(acc_ref, cnt_ref, x_ref, wl_ref, wr_ref, b_ref, o_ref):
    a = acc_ref[0] + acc_ref[1]
    cn = jnp.max(cnt_ref[0] + cnt_ref[1], axis=1, keepdims=True)
    mean = a / jnp.clip(cn, 1.0, None)
    o = (jnp.dot(mean, wl_ref[...], preferred_element_type=jnp.float32)
         + b_ref[...]
         + jnp.dot(x_ref[...], wr_ref[...], preferred_element_type=jnp.float32))
    if relu:
      o = jnp.maximum(o, 0.0)
    o_ref[...] = o

  return pl.pallas_call(
      body,
      grid=(N_PAD // R,),
      in_specs=[
          pl.BlockSpec((2, R, D), lambda i: (0, i, 0)),
          pl.BlockSpec((2, R, CW), lambda i: (0, i, 0)),
          pl.BlockSpec((R, D), lambda i: (i, 0)),
          pl.BlockSpec((D, D), lambda i: (0, 0)),
          pl.BlockSpec((D, D), lambda i: (0, 0)),
          pl.BlockSpec((1, D), lambda i: (0, 0)),
      ],
      out_specs=pl.BlockSpec((R, D), lambda i: (i, 0)),
      out_shape=jax.ShapeDtypeStruct((N_PAD, D), jnp.float32),
  )


_dense_relu = _make_dense(True)
_dense_lin = _make_dense(False)


def kernel(x, edge_index, W_l1, W_r1, b1, W_l2, W_r2, b2):
  src = edge_index[0].astype(jnp.int32)
  dst = edge_index[1].astype(jnp.int32)
  src_p = jnp.pad(src, (0, E_PAD - N_EDGES)).reshape(NW, NCHUNK, C)
  dst_p = jnp.pad(dst, (0, E_PAD - N_EDGES),
                  constant_values=N_PAD - 1).reshape(NW, NCHUNK, C)
  x_p = jnp.pad(x, ((0, N_PAD - N_NODES), (0, 0)))

  acc1, cnt = _agg_count(x_p, src_p, dst_p)
  h = _dense_relu(acc1, cnt, x_p, W_l1, W_r1, b1.reshape(1, D))
  acc2 = _agg_plain(h, src_p, dst_p)
  out = _dense_lin(acc2, cnt, h, W_l2, W_r2, b2.reshape(1, D))
  return out[:N_NODES]


# speedup vs baseline: 5.3611x; 5.3611x over previous
import functools

import jax
import jax.numpy as jnp
from jax import lax
from jax.experimental import pallas as pl
from jax.experimental.pallas import tpu as pltpu
from jax.experimental.pallas import tpu_sc as plsc

N_NODES = 10000
N_EDGES = 320000
D = 128
DP = D // 2
NC, NS = 2, 16
NW = NC * NS
C = 128
NCHUNK = 80
EPW = NCHUNK * C
E_PAD = NW * EPW
N_PAD = 10112
RPT = N_PAD // NS
_HI = -65536


@functools.cache
def _make_agg():
  mesh = plsc.VectorSubcoreMesh(core_axis_name="c", subcore_axis_name="s")
  out_type = jax.ShapeDtypeStruct((NC, N_PAD, D), jnp.float32)
  scratch = [
      pltpu.VMEM((C,), jnp.int32),
      pltpu.VMEM((C,), jnp.int32),
      pltpu.VMEM((C, DP), jnp.int32),
      pltpu.VMEM((C, DP), jnp.int32),
      pltpu.VMEM((C, D), jnp.float32),
      pltpu.VMEM((NCHUNK, C), jnp.int32),
      pltpu.VMEM_SHARED((N_PAD, D), jnp.float32),
      pltpu.SemaphoreType.DMA,
      pltpu.SemaphoreType.DMA,
      pltpu.SemaphoreType.DMA,
      pltpu.SemaphoreType.DMA,
  ]

  def body(x_hbm, src_hbm, dst_hbm, acc_out, s0, s1, pk0, pk1, rows_f,
           dst_all, acc_sh, semi0, semi1, semg0, semg1):
    _ZERO16 = jnp.zeros((16,), jnp.float32)
    c = lax.axis_index("c")
    s = lax.axis_index("s")
    w = c * NS + s
    base = s * RPT

    def zrow_body(i, carry):
      for j in range(D // 16):
        rows_f[i, pl.ds(j * 16, 16)] = _ZERO16
      return carry

    lax.fori_loop(0, C, zrow_body, 0)
    for k in range(RPT // C):
      pltpu.sync_copy(rows_f, acc_sh.at[pl.ds(base + k * C, C)])
    rem = RPT - (RPT // C) * C
    if rem:
      pltpu.sync_copy(rows_f.at[pl.ds(0, rem)],
                      acc_sh.at[pl.ds(base + (RPT // C) * C, rem)])
    pltpu.sync_copy(dst_hbm.at[w], dst_all)
    plsc.subcore_barrier()

    def widen(pk):
      def conv_body(i, carry):
        for g in range(DP // 16):
          v = pk[i, pl.ds(16 * g, 16)]
          rows_f[i, pl.ds(16 * g, 16)] = plsc.bitcast(
              v << 16, jnp.float32)
          rows_f[i, pl.ds(DP + 16 * g, 16)] = plsc.bitcast(
              v & _HI, jnp.float32)
        return carry

      lax.fori_loop(0, C, conv_body, 0)

    pltpu.sync_copy(src_hbm.at[w, 0], s0)
    pltpu.async_copy(x_hbm.at[s0], pk0, semg0)
    pltpu.async_copy(src_hbm.at[w, 1], s1, semi1)

    def pair_body(k, carry):
      j0 = 2 * k
      j1 = j0 + 1
      pltpu.make_async_copy(x_hbm.at[s0], pk0, semg0).wait()

      @pl.when(j0 + 2 < NCHUNK)
      def _():
        pltpu.async_copy(src_hbm.at[w, j0 + 2], s0, semi0)

      pltpu.make_async_copy(src_hbm.at[w, j1], s1, semi1).wait()
      pltpu.async_copy(x_hbm.at[s1], pk1, semg1)
      widen(pk0)
      pltpu.sync_copy(rows_f, acc_sh.at[dst_all.at[j0]], add=True)

      pltpu.make_async_copy(x_hbm.at[s1], pk1, semg1).wait()

      @pl.when(j1 + 2 < NCHUNK)
      def _():
        pltpu.async_copy(src_hbm.at[w, j1 + 2], s1, semi1)

      @pl.when(j0 + 2 < NCHUNK)
      def _():
        pltpu.make_async_copy(src_hbm.at[w, j0 + 2], s0, semi0).wait()
        pltpu.async_copy(x_hbm.at[s0], pk0, semg0)

      widen(pk1)
      pltpu.sync_copy(rows_f, acc_sh.at[dst_all.at[j1]], add=True)
      return carry

    lax.fori_loop(0, NCHUNK // 2, pair_body, 0)
    plsc.subcore_barrier()

    pltpu.sync_copy(acc_sh.at[pl.ds(base, RPT)], acc_out.at[c, pl.ds(base, RPT)])

  return pl.kernel(body, mesh=mesh, out_type=out_type,
                   scratch_types=tuple(scratch),
                   compiler_params=pltpu.CompilerParams(
                       needs_layout_passes=False,
                       use_tc_tiling_on_sc=False))


HR = 80
EB = 2048


def _hist_body(dst_ref, o_ref):
  i = pl.program_id(0)

  @pl.when(i == 0)
  def _():
    o_ref[...] = jnp.zeros_like(o_ref)

  d = dst_ref[...]
  row_ids = jax.lax.broadcasted_iota(jnp.int32, (EB, HR), 1)
  col_ids = jax.lax.broadcasted_iota(jnp.int32, (EB, D), 1)
  oh_r = jnp.where(row_ids == (d >> 7), 1.0, 0.0).astype(jnp.float32)
  oh_c = jnp.where(col_ids == (d & 127), 1.0, 0.0).astype(jnp.float32)
  o_ref[...] += jax.lax.dot_general(
      oh_r, oh_c, (((0,), (0,)), ((), ())),
      preferred_element_type=jnp.float32)


def _hist(dst2):
  return pl.pallas_call(
      _hist_body,
      grid=(E_PAD // EB,),
      in_specs=[pl.BlockSpec((EB, 1), lambda i: (i, 0))],
      out_specs=pl.BlockSpec((HR, D), lambda i: (0, 0)),
      out_shape=jax.ShapeDtypeStruct((HR, D), jnp.float32),
  )(dst2)


def _pack_rows(o):
  lo = jax.lax.bitcast_convert_type(o[:, :DP], jnp.int32)
  hi = jax.lax.bitcast_convert_type(o[:, DP:], jnp.int32)
  lo16 = jax.lax.shift_right_logical(lo + 32768, 16)
  hi16 = (hi + 32768) & _HI
  return lo16 | hi16


def _unpack_rows(p):
  lo = jax.lax.bitcast_convert_type(p << 16, jnp.float32)
  hi = jax.lax.bitcast_convert_type(p & _HI, jnp.float32)
  return jnp.concatenate([lo, hi], axis=1)


R_TC = 632


def _dense1_body(acc_ref, cnt_ref, x_ref, wl_ref, wr_ref, b_ref, o_ref):
  a = acc_ref[0] + acc_ref[1]
  mean = a / jnp.clip(cnt_ref[...], 1.0, None)
  o = (jnp.dot(mean, wl_ref[...], preferred_element_type=jnp.float32)
       + b_ref[...]
       + jnp.dot(x_ref[...], wr_ref[...], preferred_element_type=jnp.float32))
  o = jnp.maximum(o, 0.0)
  o_ref[...] = _pack_rows(o)


def _dense2_body(acc_ref, cnt_ref, x_ref, wl_ref, wr_ref, b_ref, o_ref):
  a = acc_ref[0] + acc_ref[1]
  mean = a / jnp.clip(cnt_ref[...], 1.0, None)
  h = _unpack_rows(x_ref[...])
  o_ref[...] = (
      jnp.dot(mean, wl_ref[...], preferred_element_type=jnp.float32)
      + b_ref[...]
      + jnp.dot(h, wr_ref[...], preferred_element_type=jnp.float32))


def _dense(body, x_width, x_dtype, out_width, out_dtype):
  return pl.pallas_call(
      body,
      grid=(N_PAD // R_TC,),
      in_specs=[
          pl.BlockSpec((2, R_TC, D), lambda i: (0, i, 0)),
          pl.BlockSpec((R_TC, 1), lambda i: (i, 0)),
          pl.BlockSpec((R_TC, x_width), lambda i: (i, 0)),
          pl.BlockSpec((D, D), lambda i: (0, 0)),
          pl.BlockSpec((D, D), lambda i: (0, 0)),
          pl.BlockSpec((1, D), lambda i: (0, 0)),
      ],
      out_specs=pl.BlockSpec((R_TC, out_width), lambda i: (i, 0)),
      out_shape=jax.ShapeDtypeStruct((N_PAD, out_width), out_dtype),
  )


def kernel(x, edge_index, W_l1, W_r1, b1, W_l2, W_r2, b2):
  src = edge_index[0].astype(jnp.int32)
  dst = edge_index[1].astype(jnp.int32)
  src_p = jnp.pad(src, (0, E_PAD - N_EDGES)).reshape(NW, NCHUNK, C)
  dst_p = jnp.pad(dst, (0, E_PAD - N_EDGES),
                  constant_values=N_PAD - 1).reshape(NW, NCHUNK, C)
  x_p = jnp.pad(x, ((0, N_PAD - N_NODES), (0, 0)))
  xl = x_p[:, :DP].astype(jnp.bfloat16)
  xh = x_p[:, DP:].astype(jnp.bfloat16)
  xpk = jax.lax.bitcast_convert_type(jnp.stack([xl, xh], axis=-1), jnp.int32)

  x_p = pltpu.with_memory_space_constraint(x_p, pltpu.MemorySpace.HBM)
  src_p = pltpu.with_memory_space_constraint(src_p, pltpu.MemorySpace.HBM)
  dst_p = pltpu.with_memory_space_constraint(dst_p, pltpu.MemorySpace.HBM)
  xpk = pltpu.with_memory_space_constraint(xpk, pltpu.MemorySpace.HBM)

  dst2 = dst_p.reshape(E_PAD, 1)
  cnt = _hist(dst2).reshape(HR * D, 1)[:N_PAD]

  agg = _make_agg()
  acc1 = agg(xpk, src_p, dst_p)
  hpk = _dense(_dense1_body, D, jnp.float32, DP, jnp.int32)(
      acc1, cnt, x_p, W_l1, W_r1, b1.reshape(1, D))
  hpk = pltpu.with_memory_space_constraint(hpk, pltpu.MemorySpace.HBM)
  acc2 = agg(hpk, src_p, dst_p)
  out = _dense(_dense2_body, DP, jnp.int32, D, jnp.float32)(
      acc2, cnt, hpk, W_l2, W_r2, b2.reshape(1, D))
  return out[:N_NODES]
